# Initial kernel scaffold; baseline (speedup 1.0000x reference)
import jax
import jax.numpy as jnp
from jax.experimental import pallas as pl


def kernel(x, W):
    raise NotImplementedError("write your pallas kernel here")


# speedup vs baseline: 1.4161x; 1.4161x over previous
import functools

import jax
import jax.numpy as jnp
from jax import lax
from jax.experimental import pallas as pl
from jax.experimental.pallas import tpu as pltpu
from jax.experimental.pallas import tpu_sc as plsc

NUM_CORES = 2
NUM_SUBCORES = 16
NUM_WORKERS = NUM_CORES * NUM_SUBCORES


def _make_lookup(B, V, D, chunk):
    b_per_w = B // NUM_WORKERS
    n_chunks = b_per_w // chunk
    assert b_per_w % chunk == 0 and n_chunks % 2 == 0 and B % NUM_WORKERS == 0

    mesh = plsc.VectorSubcoreMesh(
        core_axis_name="c",
        subcore_axis_name="s",
        num_cores=NUM_CORES,
        num_subcores=NUM_SUBCORES,
    )

    @functools.partial(
        pl.kernel,
        out_type=jax.ShapeDtypeStruct((B, D), jnp.float32),
        mesh=mesh,
        compiler_params=pltpu.CompilerParams(use_tc_tiling_on_sc=False),
        scratch_types=[
            pltpu.VMEM((b_per_w,), jnp.int32),
            pltpu.VMEM((chunk, D), jnp.float32),
            pltpu.VMEM((chunk, D), jnp.float32),
            pltpu.SemaphoreType.DMA,
            pltpu.SemaphoreType.DMA,
            pltpu.SemaphoreType.DMA,
            pltpu.SemaphoreType.DMA,
        ],
    )
    def lookup(idx_hbm, w_hbm, out_hbm, idx_v, buf0, buf1, gs0, gs1, ws0, ws1):
        wid = lax.axis_index("s") * NUM_CORES + lax.axis_index("c")
        base = wid * b_per_w

        pltpu.sync_copy(idx_hbm.at[pl.ds(base, b_per_w)], idx_v)

        def gather(g, buf, sem):
            return pltpu.async_copy(
                w_hbm.at[idx_v.at[pl.ds(g * chunk, chunk)]], buf, sem
            )

        def write(g, buf, sem):
            return pltpu.async_copy(
                buf, out_hbm.at[pl.ds(base + g * chunk, chunk)], sem
            )

        g0 = gather(0, buf0, gs0)
        g1 = gather(1, buf1, gs1)
        g0.wait()
        write(0, buf0, ws0)
        g1.wait()
        write(1, buf1, ws1)

        @pl.loop(2, n_chunks, step=2)
        def _(g):
            pltpu.make_async_copy(
                buf0, out_hbm.at[pl.ds(base + (g - 2) * chunk, chunk)], ws0
            ).wait()
            ga = gather(g, buf0, gs0)
            pltpu.make_async_copy(
                buf1, out_hbm.at[pl.ds(base + (g - 1) * chunk, chunk)], ws1
            ).wait()
            gb = gather(g + 1, buf1, gs1)
            ga.wait()
            write(g, buf0, ws0)
            gb.wait()
            write(g + 1, buf1, ws1)

        pltpu.make_async_copy(
            buf0, out_hbm.at[pl.ds(base + (n_chunks - 2) * chunk, chunk)], ws0
        ).wait()
        pltpu.make_async_copy(
            buf1, out_hbm.at[pl.ds(base + (n_chunks - 1) * chunk, chunk)], ws1
        ).wait()

    return lookup


def kernel(x, W):
    B, H = x.shape
    V, D = W.shape
    flat = x.reshape(-1).astype(jnp.int32)
    out = _make_lookup(B * H, V, D, chunk=40)(flat, W)
    return out.reshape(B, H, D)
